# baseline (speedup 1.0000x reference)
import functools

import jax
import jax.numpy as jnp
from jax import lax
from jax.experimental import pallas as pl
from jax.experimental.pallas import tpu as pltpu
from jax.experimental.pallas import tpu_sc as plsc

N = 10000
F = 128
N_EDGES = 320000
NP = 10240
CHUNK = 128
NC = 2
NS = 16
NW = NC * NS
C_PER_SUB = 80
C2 = NW * C_PER_SUB
E_PAD = C2 * CHUNK
ROWS_PER_SUB = NP // NS
OUT_BIG = 632
OUT_LAST = N - (NS - 1) * OUT_BIG

_MESH = plsc.VectorSubcoreMesh(core_axis_name="c", subcore_axis_name="s",
                               num_cores=NC, num_subcores=NS)


@functools.partial(
    pl.kernel, mesh=_MESH,
    out_type=jax.ShapeDtypeStruct((NC, NP), jnp.float32),
    scratch_types=[
        pltpu.VMEM((C_PER_SUB, CHUNK), jnp.int32),
        pltpu.VMEM((CHUNK,), jnp.float32),
        pltpu.VMEM_SHARED((NP,), jnp.float32),
        pltpu.SemaphoreType.DMA,
    ],
)
def _sc_deg(dst_hbm, zeros1_hbm, out_hbm, didx, ones, acc, sem):
    c = lax.axis_index("c")
    s = lax.axis_index("s")
    w = c * NS + s
    for i in range(CHUNK // 16):
        ones[pl.ds(i * 16, 16)] = jnp.ones((16,), jnp.float32)
    pltpu.sync_copy(dst_hbm.at[pl.ds(w * C_PER_SUB, C_PER_SUB)], didx)
    pltpu.sync_copy(zeros1_hbm.at[pl.ds(s * ROWS_PER_SUB, ROWS_PER_SUB)],
                    acc.at[pl.ds(s * ROWS_PER_SUB, ROWS_PER_SUB)])
    plsc.subcore_barrier()

    LAG = 8

    def body(i, _):
        pltpu.async_copy(ones, acc.at[didx.at[i]], sem, add=True)

        @pl.when(i >= LAG)
        def _():
            pltpu.make_async_copy(ones, acc.at[didx.at[i - LAG]], sem).wait()

        return 0

    lax.fori_loop(0, C_PER_SUB, body, 0)

    def drain(i, _):
        pltpu.make_async_copy(ones, acc.at[didx.at[i]], sem).wait()
        return 0

    lax.fori_loop(C_PER_SUB - LAG, C_PER_SUB, drain, 0)
    plsc.subcore_barrier()
    pltpu.sync_copy(acc.at[pl.ds(s * ROWS_PER_SUB, ROWS_PER_SUB)],
                    out_hbm.at[c, pl.ds(s * ROWS_PER_SUB, ROWS_PER_SUB)])


@functools.partial(
    pl.kernel, mesh=_MESH,
    out_type=jax.ShapeDtypeStruct((NC, N, F), jnp.float32),
    scratch_types=[
        pltpu.VMEM((C_PER_SUB // 2, CHUNK), jnp.int32),
        pltpu.VMEM((C_PER_SUB // 2, CHUNK), jnp.int32),
        pltpu.VMEM((CHUNK, F), jnp.float32),
        pltpu.VMEM((CHUNK, F), jnp.float32),
        pltpu.VMEM_SHARED((NP, F), jnp.float32),
        pltpu.SemaphoreType.DMA,
        pltpu.SemaphoreType.DMA,
    ],
)
def _sc_agg(src_hbm, dst_hbm, tab_hbm, zeros_hbm, out_hbm,
            sidx, didx, rows0, rows1, acc, gsem0, gsem1):
    c = lax.axis_index("c")
    s = lax.axis_index("s")
    w = c * NS + s
    @pl.when(s < NS - 1)
    def _():
        pltpu.sync_copy(zeros_hbm.at[pl.ds(s * OUT_BIG, OUT_BIG)],
                        acc.at[pl.ds(s * OUT_BIG, OUT_BIG)])

    @pl.when(s == NS - 1)
    def _():
        pltpu.sync_copy(zeros_hbm.at[pl.ds(s * OUT_BIG, OUT_LAST)],
                        acc.at[pl.ds(s * OUT_BIG, OUT_LAST)])

    plsc.subcore_barrier()

    CP = C_PER_SUB // 2
    n_outer = CP // 2
    for p in range(2):
        base = w * C_PER_SUB + p * CP
        pltpu.sync_copy(src_hbm.at[pl.ds(base, CP)], sidx)
        pltpu.sync_copy(dst_hbm.at[pl.ds(base, CP)], didx)
        pltpu.async_copy(tab_hbm.at[sidx.at[0]], rows0, gsem0)

        def body(k, _):
            i = 2 * k
            d1 = pltpu.async_copy(tab_hbm.at[sidx.at[i + 1]], rows1, gsem1)
            pltpu.make_async_copy(tab_hbm.at[sidx.at[i]], rows0, gsem0).wait()
            pltpu.sync_copy(rows0, acc.at[didx.at[i]], add=True)

            @pl.when(k < n_outer - 1)
            def _():
                pltpu.async_copy(tab_hbm.at[sidx.at[i + 2]], rows0, gsem0)

            d1.wait()
            pltpu.sync_copy(rows1, acc.at[didx.at[i + 1]], add=True)
            return 0

        lax.fori_loop(0, n_outer, body, 0)
    plsc.subcore_barrier()

    @pl.when(s < NS - 1)
    def _():
        pltpu.sync_copy(acc.at[pl.ds(s * OUT_BIG, OUT_BIG)],
                        out_hbm.at[c, pl.ds(s * OUT_BIG, OUT_BIG)])

    @pl.when(s == NS - 1)
    def _():
        pltpu.sync_copy(acc.at[pl.ds(s * OUT_BIG, OUT_LAST)],
                        out_hbm.at[c, pl.ds(s * OUT_BIG, OUT_LAST)])


BLK = 1000
GRID = N // BLK


def _tc_mm_body(x_ref, w_ref, o_ref):
    o_ref[...] = jnp.dot(x_ref[...], w_ref[...],
                         preferred_element_type=jnp.float32)


def _tc_mm(x, W1):
    return pl.pallas_call(
        _tc_mm_body,
        grid=(GRID,),
        in_specs=[
            pl.BlockSpec((BLK, F), lambda i: (i, 0)),
            pl.BlockSpec((F, F), lambda i: (0, 0)),
        ],
        out_specs=pl.BlockSpec((BLK, F), lambda i: (i, 0)),
        out_shape=jax.ShapeDtypeStruct((N, F), jnp.float32),
    )(x, W1)


def _tc_scale_body(t_ref, dg_ref, o_ref):
    o_ref[...] = t_ref[...] * lax.rsqrt(dg_ref[...])


def _tc_scale(t, dg):
    return pl.pallas_call(
        _tc_scale_body,
        grid=(GRID,),
        in_specs=[
            pl.BlockSpec((BLK, F), lambda i: (i, 0)),
            pl.BlockSpec((BLK, 1), lambda i: (i, 0)),
        ],
        out_specs=pl.BlockSpec((BLK, F), lambda i: (i, 0)),
        out_shape=jax.ShapeDtypeStruct((N, F), jnp.float32),
    )(t, dg)


def _tc2_body(p0_ref, p1_ref, g1_ref, dg_ref, b_ref, gm_ref, bt_ref, o_ref):
    dinv = lax.rsqrt(dg_ref[...])
    hpre = (p0_ref[...] + p1_ref[...] + g1_ref[...]) * dinv + b_ref[...][None, :]
    h = jnp.maximum(hpre, 0.0)
    mu = jnp.mean(h, axis=1, keepdims=True)
    var = jnp.mean((h - mu) * (h - mu), axis=1, keepdims=True)
    hn = (h - mu) * lax.rsqrt(var + 1e-5) * gm_ref[...][None, :] + bt_ref[...][None, :]
    o_ref[...] = hn * dinv


def _tc2(p0, p1, g1, dg, b1, gamma, beta):
    return pl.pallas_call(
        _tc2_body,
        grid=(GRID,),
        in_specs=[
            pl.BlockSpec((BLK, F), lambda i: (i, 0)),
            pl.BlockSpec((BLK, F), lambda i: (i, 0)),
            pl.BlockSpec((BLK, F), lambda i: (i, 0)),
            pl.BlockSpec((BLK, 1), lambda i: (i, 0)),
            pl.BlockSpec((F,), lambda i: (0,)),
            pl.BlockSpec((F,), lambda i: (0,)),
            pl.BlockSpec((F,), lambda i: (0,)),
        ],
        out_specs=pl.BlockSpec((BLK, F), lambda i: (i, 0)),
        out_shape=jax.ShapeDtypeStruct((N, F), jnp.float32),
    )(p0, p1, g1, dg, b1, gamma, beta)


def _tc3_body(q0_ref, q1_ref, g2_ref, dg_ref, w2_ref, w3_ref, b2_ref, b3_ref,
              mu_ref, lv_ref):
    ph = (q0_ref[...] + q1_ref[...] + g2_ref[...]) * lax.rsqrt(dg_ref[...])
    mu_ref[...] = (jnp.dot(ph, w2_ref[...], preferred_element_type=jnp.float32)
                   + b2_ref[...][None, :])
    lv_ref[...] = (jnp.dot(ph, w3_ref[...], preferred_element_type=jnp.float32)
                   + b3_ref[...][None, :])


def _tc3(q0, q1, g2, dg, W2, W3, b2, b3):
    return pl.pallas_call(
        _tc3_body,
        grid=(GRID,),
        in_specs=[
            pl.BlockSpec((BLK, F), lambda i: (i, 0)),
            pl.BlockSpec((BLK, F), lambda i: (i, 0)),
            pl.BlockSpec((BLK, F), lambda i: (i, 0)),
            pl.BlockSpec((BLK, 1), lambda i: (i, 0)),
            pl.BlockSpec((F, 64), lambda i: (0, 0)),
            pl.BlockSpec((F, 64), lambda i: (0, 0)),
            pl.BlockSpec((64,), lambda i: (0,)),
            pl.BlockSpec((64,), lambda i: (0,)),
        ],
        out_specs=[
            pl.BlockSpec((BLK, 64), lambda i: (i, 0)),
            pl.BlockSpec((BLK, 64), lambda i: (i, 0)),
        ],
        out_shape=[
            jax.ShapeDtypeStruct((N, 64), jnp.float32),
            jax.ShapeDtypeStruct((N, 64), jnp.float32),
        ],
    )(q0, q1, g2, dg, W2, W3, b2, b3)


def kernel(x, edge_index, W1, b1, gamma, beta, W2, b2, W3, b3):
    n_dummy = E_PAD - N_EDGES
    ar = jnp.arange(n_dummy, dtype=jnp.int32)
    pad_src = ar % N
    pad_dst = ar % (NP - N) + N
    src2d = jnp.concatenate([edge_index[0], pad_src]).reshape(C2, CHUNK)
    dst2d = jnp.concatenate([edge_index[1], pad_dst]).reshape(C2, CHUNK)
    zeros1 = jnp.zeros((NP,), jnp.float32)
    zeros2 = jnp.zeros((N, F), jnp.float32)

    t = _tc_mm(x, W1)
    degp = _sc_deg(dst2d, zeros1)
    dg = (degp[0, :N] + degp[1, :N] + 1.0)[:, None]

    g1 = _tc_scale(t, dg)
    p = _sc_agg(src2d, dst2d, g1, zeros2)
    g2 = _tc2(p[0], p[1], g1, dg, b1, gamma, beta)
    q = _sc_agg(src2d, dst2d, g2, zeros2)

    mu, logvar = _tc3(q[0], q[1], g2, dg, W2, W3, b2, b3)
    return mu, logvar


# speedup vs baseline: 1.0075x; 1.0075x over previous
import functools

import jax
import jax.numpy as jnp
from jax import lax
from jax.experimental import pallas as pl
from jax.experimental.pallas import tpu as pltpu
from jax.experimental.pallas import tpu_sc as plsc

N = 10000
F = 128
N_EDGES = 320000
NP = 10240
CHUNK = 128
NC = 2
NS = 16
NW = NC * NS
C_PER_SUB = 80
C2 = NW * C_PER_SUB
E_PAD = C2 * CHUNK
ROWS_PER_SUB = NP // NS
OUT_BIG = 632
OUT_LAST = N - (NS - 1) * OUT_BIG

_MESH = plsc.VectorSubcoreMesh(core_axis_name="c", subcore_axis_name="s",
                               num_cores=NC, num_subcores=NS)


@functools.partial(
    pl.kernel, mesh=_MESH,
    out_type=jax.ShapeDtypeStruct((NC, NP), jnp.float32),
    scratch_types=[
        pltpu.VMEM((C_PER_SUB, CHUNK), jnp.int32),
        pltpu.VMEM((CHUNK,), jnp.float32),
        pltpu.VMEM_SHARED((NP,), jnp.float32),
        pltpu.SemaphoreType.DMA,
    ],
)
def _sc_deg(dst_hbm, zeros1_hbm, out_hbm, didx, ones, acc, sem):
    c = lax.axis_index("c")
    s = lax.axis_index("s")
    w = c * NS + s
    for i in range(CHUNK // 16):
        ones[pl.ds(i * 16, 16)] = jnp.ones((16,), jnp.float32)
    pltpu.sync_copy(dst_hbm.at[pl.ds(w * C_PER_SUB, C_PER_SUB)], didx)
    pltpu.sync_copy(zeros1_hbm.at[pl.ds(s * ROWS_PER_SUB, ROWS_PER_SUB)],
                    acc.at[pl.ds(s * ROWS_PER_SUB, ROWS_PER_SUB)])
    plsc.subcore_barrier()

    LAG = 8

    def body(i, _):
        pltpu.async_copy(ones, acc.at[didx.at[i]], sem, add=True)

        @pl.when(i >= LAG)
        def _():
            pltpu.make_async_copy(ones, acc.at[didx.at[i - LAG]], sem).wait()

        return 0

    lax.fori_loop(0, C_PER_SUB, body, 0)

    def drain(i, _):
        pltpu.make_async_copy(ones, acc.at[didx.at[i]], sem).wait()
        return 0

    lax.fori_loop(C_PER_SUB - LAG, C_PER_SUB, drain, 0)
    plsc.subcore_barrier()
    pltpu.sync_copy(acc.at[pl.ds(s * ROWS_PER_SUB, ROWS_PER_SUB)],
                    out_hbm.at[c, pl.ds(s * ROWS_PER_SUB, ROWS_PER_SUB)])


@functools.partial(
    pl.kernel, mesh=_MESH,
    out_type=jax.ShapeDtypeStruct((NC, N, F), jnp.float32),
    scratch_types=[
        pltpu.VMEM((C_PER_SUB // 2, CHUNK), jnp.int32),
        pltpu.VMEM((C_PER_SUB // 2, CHUNK), jnp.int32),
        pltpu.VMEM((CHUNK, F), jnp.float32),
        pltpu.VMEM((CHUNK, F), jnp.float32),
        pltpu.VMEM_SHARED((NP, F), jnp.float32),
        pltpu.SemaphoreType.DMA,
        pltpu.SemaphoreType.DMA,
    ],
)
def _sc_agg(src_hbm, dst_hbm, tab_hbm, zeros_hbm, out_hbm,
            sidx, didx, rows0, rows1, acc, gsem0, gsem1):
    c = lax.axis_index("c")
    s = lax.axis_index("s")
    w = c * NS + s
    @pl.when(s < NS - 1)
    def _():
        pltpu.sync_copy(zeros_hbm.at[pl.ds(s * OUT_BIG, OUT_BIG)],
                        acc.at[pl.ds(s * OUT_BIG, OUT_BIG)])

    @pl.when(s == NS - 1)
    def _():
        pltpu.sync_copy(zeros_hbm.at[pl.ds(s * OUT_BIG, OUT_LAST)],
                        acc.at[pl.ds(s * OUT_BIG, OUT_LAST)])

    plsc.subcore_barrier()

    CP = C_PER_SUB // 2
    n_outer = CP // 2
    for p in range(2):
        base = w * C_PER_SUB + p * CP
        pltpu.sync_copy(src_hbm.at[pl.ds(base, CP)], sidx)
        pltpu.sync_copy(dst_hbm.at[pl.ds(base, CP)], didx)
        pltpu.async_copy(tab_hbm.at[sidx.at[0]], rows0, gsem0)

        def body(k, _):
            i = 2 * k
            d1 = pltpu.async_copy(tab_hbm.at[sidx.at[i + 1]], rows1, gsem1)
            pltpu.make_async_copy(tab_hbm.at[sidx.at[i]], rows0, gsem0).wait()
            pltpu.sync_copy(rows0, acc.at[didx.at[i]], add=True)

            @pl.when(k < n_outer - 1)
            def _():
                pltpu.async_copy(tab_hbm.at[sidx.at[i + 2]], rows0, gsem0)

            d1.wait()
            pltpu.sync_copy(rows1, acc.at[didx.at[i + 1]], add=True)
            return 0

        lax.fori_loop(0, n_outer, body, 0)
    plsc.subcore_barrier()

    @pl.when(s < NS - 1)
    def _():
        pltpu.sync_copy(acc.at[pl.ds(s * OUT_BIG, OUT_BIG)],
                        out_hbm.at[c, pl.ds(s * OUT_BIG, OUT_BIG)])

    @pl.when(s == NS - 1)
    def _():
        pltpu.sync_copy(acc.at[pl.ds(s * OUT_BIG, OUT_LAST)],
                        out_hbm.at[c, pl.ds(s * OUT_BIG, OUT_LAST)])


BLK = 1000
GRID = N // BLK


def _tc1_body(x_ref, w_ref, dg_ref, o_ref):
    t = jnp.dot(x_ref[...], w_ref[...], preferred_element_type=jnp.float32)
    o_ref[...] = t * lax.rsqrt(dg_ref[...])


def _tc1(x, W1, dg):
    return pl.pallas_call(
        _tc1_body,
        grid=(GRID,),
        in_specs=[
            pl.BlockSpec((BLK, F), lambda i: (i, 0)),
            pl.BlockSpec((F, F), lambda i: (0, 0)),
            pl.BlockSpec((BLK, 1), lambda i: (i, 0)),
        ],
        out_specs=pl.BlockSpec((BLK, F), lambda i: (i, 0)),
        out_shape=jax.ShapeDtypeStruct((N, F), jnp.float32),
    )(x, W1, dg)


def _tc2_body(p0_ref, p1_ref, g1_ref, dg_ref, b_ref, gm_ref, bt_ref, o_ref):
    dinv = lax.rsqrt(dg_ref[...])
    hpre = (p0_ref[...] + p1_ref[...] + g1_ref[...]) * dinv + b_ref[...][None, :]
    h = jnp.maximum(hpre, 0.0)
    mu = jnp.mean(h, axis=1, keepdims=True)
    var = jnp.mean((h - mu) * (h - mu), axis=1, keepdims=True)
    hn = (h - mu) * lax.rsqrt(var + 1e-5) * gm_ref[...][None, :] + bt_ref[...][None, :]
    o_ref[...] = hn * dinv


def _tc2(p0, p1, g1, dg, b1, gamma, beta):
    return pl.pallas_call(
        _tc2_body,
        grid=(GRID,),
        in_specs=[
            pl.BlockSpec((BLK, F), lambda i: (i, 0)),
            pl.BlockSpec((BLK, F), lambda i: (i, 0)),
            pl.BlockSpec((BLK, F), lambda i: (i, 0)),
            pl.BlockSpec((BLK, 1), lambda i: (i, 0)),
            pl.BlockSpec((F,), lambda i: (0,)),
            pl.BlockSpec((F,), lambda i: (0,)),
            pl.BlockSpec((F,), lambda i: (0,)),
        ],
        out_specs=pl.BlockSpec((BLK, F), lambda i: (i, 0)),
        out_shape=jax.ShapeDtypeStruct((N, F), jnp.float32),
    )(p0, p1, g1, dg, b1, gamma, beta)


def _tc3_body(q0_ref, q1_ref, g2_ref, dg_ref, w2_ref, w3_ref, b2_ref, b3_ref,
              mu_ref, lv_ref):
    ph = (q0_ref[...] + q1_ref[...] + g2_ref[...]) * lax.rsqrt(dg_ref[...])
    mu_ref[...] = (jnp.dot(ph, w2_ref[...], preferred_element_type=jnp.float32)
                   + b2_ref[...][None, :])
    lv_ref[...] = (jnp.dot(ph, w3_ref[...], preferred_element_type=jnp.float32)
                   + b3_ref[...][None, :])


def _tc3(q0, q1, g2, dg, W2, W3, b2, b3):
    return pl.pallas_call(
        _tc3_body,
        grid=(GRID,),
        in_specs=[
            pl.BlockSpec((BLK, F), lambda i: (i, 0)),
            pl.BlockSpec((BLK, F), lambda i: (i, 0)),
            pl.BlockSpec((BLK, F), lambda i: (i, 0)),
            pl.BlockSpec((BLK, 1), lambda i: (i, 0)),
            pl.BlockSpec((F, 64), lambda i: (0, 0)),
            pl.BlockSpec((F, 64), lambda i: (0, 0)),
            pl.BlockSpec((64,), lambda i: (0,)),
            pl.BlockSpec((64,), lambda i: (0,)),
        ],
        out_specs=[
            pl.BlockSpec((BLK, 64), lambda i: (i, 0)),
            pl.BlockSpec((BLK, 64), lambda i: (i, 0)),
        ],
        out_shape=[
            jax.ShapeDtypeStruct((N, 64), jnp.float32),
            jax.ShapeDtypeStruct((N, 64), jnp.float32),
        ],
    )(q0, q1, g2, dg, W2, W3, b2, b3)


def kernel(x, edge_index, W1, b1, gamma, beta, W2, b2, W3, b3):
    n_dummy = E_PAD - N_EDGES
    ar = jnp.arange(n_dummy, dtype=jnp.int32)
    pad_src = ar % N
    pad_dst = ar % (NP - N) + N
    src2d = jnp.concatenate([edge_index[0], pad_src]).reshape(C2, CHUNK)
    dst2d = jnp.concatenate([edge_index[1], pad_dst]).reshape(C2, CHUNK)
    zeros1 = jnp.zeros((NP,), jnp.float32)
    zeros2 = jnp.zeros((N, F), jnp.float32)

    degp = _sc_deg(dst2d, zeros1)
    dg = (degp[0, :N] + degp[1, :N] + 1.0)[:, None]

    g1 = _tc1(x, W1, dg)
    p = _sc_agg(src2d, dst2d, g1, zeros2)
    g2 = _tc2(p[0], p[1], g1, dg, b1, gamma, beta)
    q = _sc_agg(src2d, dst2d, g2, zeros2)

    mu, logvar = _tc3(q[0], q[1], g2, dg, W2, W3, b2, b3)
    return mu, logvar


# speedup vs baseline: 1.0322x; 1.0246x over previous
import functools

import jax
import jax.numpy as jnp
from jax import lax
from jax.experimental import pallas as pl
from jax.experimental.pallas import tpu as pltpu
from jax.experimental.pallas import tpu_sc as plsc

N = 10000
F = 128
N_EDGES = 320000
NP = 10240
CHUNK = 128
NC = 2
NS = 16
NW = NC * NS
C_PER_SUB = 80
C2 = NW * C_PER_SUB
E_PAD = C2 * CHUNK
ROWS_PER_SUB = NP // NS
OUT_BIG = 632
OUT_LAST = N - (NS - 1) * OUT_BIG

_MESH = plsc.VectorSubcoreMesh(core_axis_name="c", subcore_axis_name="s",
                               num_cores=NC, num_subcores=NS)


@functools.partial(
    pl.kernel, mesh=_MESH,
    out_type=jax.ShapeDtypeStruct((NC, NP), jnp.float32),
    scratch_types=[
        pltpu.VMEM((C_PER_SUB, CHUNK), jnp.int32),
        pltpu.VMEM((CHUNK,), jnp.float32),
        pltpu.VMEM_SHARED((NP,), jnp.float32),
        pltpu.SemaphoreType.DMA,
    ],
)
def _sc_deg(dst_hbm, zeros1_hbm, out_hbm, didx, ones, acc, sem):
    c = lax.axis_index("c")
    s = lax.axis_index("s")
    w = c * NS + s
    for i in range(CHUNK // 16):
        ones[pl.ds(i * 16, 16)] = jnp.ones((16,), jnp.float32)
    pltpu.sync_copy(dst_hbm.at[pl.ds(w * C_PER_SUB, C_PER_SUB)], didx)
    pltpu.sync_copy(zeros1_hbm.at[pl.ds(s * ROWS_PER_SUB, ROWS_PER_SUB)],
                    acc.at[pl.ds(s * ROWS_PER_SUB, ROWS_PER_SUB)])
    plsc.subcore_barrier()

    LAG = 8

    def body(i, _):
        pltpu.async_copy(ones, acc.at[didx.at[i]], sem, add=True)

        @pl.when(i >= LAG)
        def _():
            pltpu.make_async_copy(ones, acc.at[didx.at[i - LAG]], sem).wait()

        return 0

    lax.fori_loop(0, C_PER_SUB, body, 0)

    def drain(i, _):
        pltpu.make_async_copy(ones, acc.at[didx.at[i]], sem).wait()
        return 0

    lax.fori_loop(C_PER_SUB - LAG, C_PER_SUB, drain, 0)
    plsc.subcore_barrier()
    pltpu.sync_copy(acc.at[pl.ds(s * ROWS_PER_SUB, ROWS_PER_SUB)],
                    out_hbm.at[c, pl.ds(s * ROWS_PER_SUB, ROWS_PER_SUB)])


@functools.partial(
    pl.kernel, mesh=_MESH,
    out_type=jax.ShapeDtypeStruct((NC, N, F), jnp.float32),
    scratch_types=[
        pltpu.VMEM((C_PER_SUB // 2, CHUNK), jnp.int32),
        pltpu.VMEM((C_PER_SUB // 2, CHUNK), jnp.int32),
        pltpu.VMEM((CHUNK, F), jnp.float32),
        pltpu.VMEM((CHUNK, F), jnp.float32),
        pltpu.VMEM_SHARED((NP, F), jnp.float32),
        pltpu.SemaphoreType.DMA,
        pltpu.SemaphoreType.DMA,
    ],
)
def _sc_agg(src_hbm, dst_hbm, tab_hbm, out_hbm,
            sidx, didx, rows0, rows1, acc, gsem0, gsem1):
    c = lax.axis_index("c")
    s = lax.axis_index("s")
    w = c * NS + s

    def zrow(r, _):
        for j in range(F // 16):
            rows0[r, pl.ds(j * 16, 16)] = jnp.zeros((16,), jnp.float32)
        return 0

    lax.fori_loop(0, CHUNK, zrow, 0)
    for j in range(ROWS_PER_SUB // CHUNK):
        pltpu.sync_copy(
            rows0, acc.at[pl.ds(s * ROWS_PER_SUB + j * CHUNK, CHUNK)])
    plsc.subcore_barrier()

    CP = C_PER_SUB // 2
    n_outer = CP // 2
    for p in range(2):
        base = w * C_PER_SUB + p * CP
        pltpu.sync_copy(src_hbm.at[pl.ds(base, CP)], sidx)
        pltpu.sync_copy(dst_hbm.at[pl.ds(base, CP)], didx)
        pltpu.async_copy(tab_hbm.at[sidx.at[0]], rows0, gsem0)

        def body(k, _):
            i = 2 * k
            d1 = pltpu.async_copy(tab_hbm.at[sidx.at[i + 1]], rows1, gsem1)
            pltpu.make_async_copy(tab_hbm.at[sidx.at[i]], rows0, gsem0).wait()
            pltpu.sync_copy(rows0, acc.at[didx.at[i]], add=True)

            @pl.when(k < n_outer - 1)
            def _():
                pltpu.async_copy(tab_hbm.at[sidx.at[i + 2]], rows0, gsem0)

            d1.wait()
            pltpu.sync_copy(rows1, acc.at[didx.at[i + 1]], add=True)
            return 0

        lax.fori_loop(0, n_outer, body, 0)
    plsc.subcore_barrier()

    @pl.when(s < NS - 1)
    def _():
        pltpu.sync_copy(acc.at[pl.ds(s * OUT_BIG, OUT_BIG)],
                        out_hbm.at[c, pl.ds(s * OUT_BIG, OUT_BIG)])

    @pl.when(s == NS - 1)
    def _():
        pltpu.sync_copy(acc.at[pl.ds(s * OUT_BIG, OUT_LAST)],
                        out_hbm.at[c, pl.ds(s * OUT_BIG, OUT_LAST)])


BLK = 1000
GRID = N // BLK


def _tc1_body(x_ref, w_ref, dg_ref, o_ref):
    t = jnp.dot(x_ref[...], w_ref[...], preferred_element_type=jnp.float32)
    o_ref[...] = t * lax.rsqrt(dg_ref[...])


def _tc1(x, W1, dg):
    return pl.pallas_call(
        _tc1_body,
        grid=(GRID,),
        in_specs=[
            pl.BlockSpec((BLK, F), lambda i: (i, 0)),
            pl.BlockSpec((F, F), lambda i: (0, 0)),
            pl.BlockSpec((BLK, 1), lambda i: (i, 0)),
        ],
        out_specs=pl.BlockSpec((BLK, F), lambda i: (i, 0)),
        out_shape=jax.ShapeDtypeStruct((N, F), jnp.float32),
    )(x, W1, dg)


def _tc2_body(p0_ref, p1_ref, g1_ref, dg_ref, b_ref, gm_ref, bt_ref, o_ref):
    dinv = lax.rsqrt(dg_ref[...])
    hpre = (p0_ref[...] + p1_ref[...] + g1_ref[...]) * dinv + b_ref[...][None, :]
    h = jnp.maximum(hpre, 0.0)
    mu = jnp.mean(h, axis=1, keepdims=True)
    var = jnp.mean((h - mu) * (h - mu), axis=1, keepdims=True)
    hn = (h - mu) * lax.rsqrt(var + 1e-5) * gm_ref[...][None, :] + bt_ref[...][None, :]
    o_ref[...] = hn * dinv


def _tc2(p0, p1, g1, dg, b1, gamma, beta):
    return pl.pallas_call(
        _tc2_body,
        grid=(GRID,),
        in_specs=[
            pl.BlockSpec((BLK, F), lambda i: (i, 0)),
            pl.BlockSpec((BLK, F), lambda i: (i, 0)),
            pl.BlockSpec((BLK, F), lambda i: (i, 0)),
            pl.BlockSpec((BLK, 1), lambda i: (i, 0)),
            pl.BlockSpec((F,), lambda i: (0,)),
            pl.BlockSpec((F,), lambda i: (0,)),
            pl.BlockSpec((F,), lambda i: (0,)),
        ],
        out_specs=pl.BlockSpec((BLK, F), lambda i: (i, 0)),
        out_shape=jax.ShapeDtypeStruct((N, F), jnp.float32),
    )(p0, p1, g1, dg, b1, gamma, beta)


def _tc3_body(q0_ref, q1_ref, g2_ref, dg_ref, w2_ref, w3_ref, b2_ref, b3_ref,
              mu_ref, lv_ref):
    ph = (q0_ref[...] + q1_ref[...] + g2_ref[...]) * lax.rsqrt(dg_ref[...])
    mu_ref[...] = (jnp.dot(ph, w2_ref[...], preferred_element_type=jnp.float32)
                   + b2_ref[...][None, :])
    lv_ref[...] = (jnp.dot(ph, w3_ref[...], preferred_element_type=jnp.float32)
                   + b3_ref[...][None, :])


def _tc3(q0, q1, g2, dg, W2, W3, b2, b3):
    return pl.pallas_call(
        _tc3_body,
        grid=(GRID,),
        in_specs=[
            pl.BlockSpec((BLK, F), lambda i: (i, 0)),
            pl.BlockSpec((BLK, F), lambda i: (i, 0)),
            pl.BlockSpec((BLK, F), lambda i: (i, 0)),
            pl.BlockSpec((BLK, 1), lambda i: (i, 0)),
            pl.BlockSpec((F, 64), lambda i: (0, 0)),
            pl.BlockSpec((F, 64), lambda i: (0, 0)),
            pl.BlockSpec((64,), lambda i: (0,)),
            pl.BlockSpec((64,), lambda i: (0,)),
        ],
        out_specs=[
            pl.BlockSpec((BLK, 64), lambda i: (i, 0)),
            pl.BlockSpec((BLK, 64), lambda i: (i, 0)),
        ],
        out_shape=[
            jax.ShapeDtypeStruct((N, 64), jnp.float32),
            jax.ShapeDtypeStruct((N, 64), jnp.float32),
        ],
    )(q0, q1, g2, dg, W2, W3, b2, b3)


def kernel(x, edge_index, W1, b1, gamma, beta, W2, b2, W3, b3):
    n_dummy = E_PAD - N_EDGES
    ar = jnp.arange(n_dummy, dtype=jnp.int32)
    pad_src = ar % N
    pad_dst = ar % (NP - N) + N
    src2d = jnp.concatenate([edge_index[0], pad_src]).reshape(C2, CHUNK)
    dst2d = jnp.concatenate([edge_index[1], pad_dst]).reshape(C2, CHUNK)
    zeros1 = jnp.zeros((NP,), jnp.float32)

    degp = _sc_deg(dst2d, zeros1)
    dg = (degp[0, :N] + degp[1, :N] + 1.0)[:, None]

    g1 = _tc1(x, W1, dg)
    p = _sc_agg(src2d, dst2d, g1)
    g2 = _tc2(p[0], p[1], g1, dg, b1, gamma, beta)
    q = _sc_agg(src2d, dst2d, g2)

    mu, logvar = _tc3(q[0], q[1], g2, dg, W2, W3, b2, b3)
    return mu, logvar
